# baseline (speedup 1.0000x reference)
import functools

import jax
import jax.numpy as jnp
from jax.experimental import pallas as pl
from jax.experimental.pallas import tpu as pltpu
from jax.experimental.pallas import tpu_sc as plsc

_NC = 2
_NS = 16
_NW = _NC * _NS
_CHUNK = 16
_NBUF = 3


def _sc_body(H, W, n, y2d_ref, x16_ref, xi_ref, yi_ref, out_ref,
             bufs, xi_v, yi_v, x_v, in_sems, out_sems):
    B_rows = y2d_ref.shape[0]
    rows_per_w = B_rows // _NW
    nchunk = rows_per_w // _CHUNK
    wid = jax.lax.axis_index("s") * _NC + jax.lax.axis_index("c")
    row0 = wid * rows_per_w

    pltpu.sync_copy(xi_ref, xi_v)
    pltpu.sync_copy(yi_ref, yi_v)
    pltpu.sync_copy(x16_ref, x_v)

    b_w = row0 // H
    vals = plsc.load_gather(x_v, [jnp.full((16,), b_w, jnp.int32)])

    nvec = n // 16
    xs = [xi_v[pl.ds(k * 16, 16)] + b_w * H for k in range(nvec)]
    ys = [yi_v[pl.ds(k * 16, 16)] for k in range(nvec)]

    def start_in(c):
        return pltpu.make_async_copy(
            y2d_ref.at[pl.ds(row0 + c * _CHUNK, _CHUNK), :],
            bufs[c % _NBUF],
            in_sems[c % _NBUF],
        )

    def start_out(c):
        return pltpu.make_async_copy(
            bufs[c % _NBUF],
            out_ref.at[pl.ds(row0 + c * _CHUNK, _CHUNK), :],
            out_sems[c % _NBUF],
        )

    start_in(0).start()
    for c in range(nchunk):
        if c + 1 < nchunk:
            if c + 1 >= _NBUF:
                start_out(c + 1 - _NBUF).wait()
            start_in(c + 1).start()
        start_in(c).wait()
        buf = bufs[c % _NBUF]
        lo = row0 + c * _CHUNK
        for k in range(nvec):
            local = xs[k] - lo
            mask = (local >= 0) & (local < _CHUNK)
            safe = jnp.clip(local, 0, _CHUNK - 1)
            plsc.addupdate_scatter(buf, [safe, ys[k]], vals, mask=mask)
        start_out(c).start()
    for c in range(max(nchunk - _NBUF, 0), nchunk):
        start_out(c).wait()


@jax.jit
def kernel(Y, X, x_idx, y_idx):
    B, H, W = Y.shape
    n = x_idx.shape[0]
    y2d = Y.reshape(B * H, W)
    x16 = jnp.pad(X.reshape(B), (0, 16 - B))
    mesh = plsc.VectorSubcoreMesh(
        core_axis_name="c", subcore_axis_name="s",
        num_cores=_NC, num_subcores=_NS,
    )
    run = pl.kernel(
        functools.partial(_sc_body, H, W, n),
        out_type=jax.ShapeDtypeStruct((B * H, W), Y.dtype),
        mesh=mesh,
        compiler_params=pltpu.CompilerParams(needs_layout_passes=False),
        scratch_types=[
            [pltpu.VMEM((_CHUNK, W), jnp.float32) for _ in range(_NBUF)],
            pltpu.VMEM((n,), jnp.int32),
            pltpu.VMEM((n,), jnp.int32),
            pltpu.VMEM((16,), jnp.float32),
            [pltpu.SemaphoreType.DMA for _ in range(_NBUF)],
            [pltpu.SemaphoreType.DMA for _ in range(_NBUF)],
        ],
    )
    out = run(y2d, x16, x_idx, y_idx)
    return out.reshape(B, H, W)


# speedup vs baseline: 1.3429x; 1.3429x over previous
import jax
import jax.numpy as jnp
from jax.experimental import pallas as pl
from jax.experimental.pallas import tpu as pltpu

_BR = 1024


def _body(x_ref, xs_ref, ys_ref, y_ref, out_ref):
    b = pl.program_id(0)
    r = pl.program_id(1)
    row_start = r * _BR
    n = xs_ref.shape[0]
    W = out_ref.shape[2]

    out_ref[0] = y_ref[0]

    xb = x_ref[b, 0]
    col = jax.lax.broadcasted_iota(jnp.int32, (1, W), 1)

    def point(i, carry):
        local = xs_ref[i] - row_start
        yi = ys_ref[i]

        @pl.when((local >= 0) & (local < _BR))
        def _():
            row = out_ref[0, pl.ds(local, 1), :]
            out_ref[0, pl.ds(local, 1), :] = row + jnp.where(col == yi, xb, 0.0)

        return carry

    jax.lax.fori_loop(0, n, point, 0)


@jax.jit
def kernel(Y, X, x_idx, y_idx):
    B, H, W = Y.shape
    return pl.pallas_call(
        _body,
        grid=(B, H // _BR),
        in_specs=[
            pl.BlockSpec(memory_space=pltpu.SMEM),
            pl.BlockSpec(memory_space=pltpu.SMEM),
            pl.BlockSpec(memory_space=pltpu.SMEM),
            pl.BlockSpec((1, _BR, W), lambda b, r: (b, r, 0)),
        ],
        out_specs=pl.BlockSpec((1, _BR, W), lambda b, r: (b, r, 0)),
        out_shape=jax.ShapeDtypeStruct((B, H, W), Y.dtype),
        compiler_params=pltpu.CompilerParams(
            dimension_semantics=("parallel", "parallel"),
        ),
    )(X, x_idx, y_idx, Y)
